# baseline (speedup 1.0000x reference)
import functools

import jax
import jax.numpy as jnp
from jax import lax
from jax.experimental import pallas as pl
from jax.experimental.pallas import tpu as pltpu
from jax.experimental.pallas import tpu_sc as plsc

K_CODES = 8192
DIM = 64
ROWS = 9216
R_BLK = 1152
C_CH = 512
LANES = 128


def _dist_argmin_body(flat_ref, cb_hbm, cn_ref, ids_ref, loss_ref,
                      cb_vmem, cb_sem, acc_ref, vacc_ref, cacc_ref):
    i = pl.program_id(0)

    @pl.when(i == 0)
    def _():
        pltpu.make_async_copy(cb_hbm, cb_vmem, cb_sem).start()
        pltpu.make_async_copy(cb_hbm, cb_vmem, cb_sem).wait()
        acc_ref[0] = 0.0

    flat = flat_ref[...]
    rn = jnp.sum(flat ** 2, axis=1, keepdims=True)
    vacc_ref[...] = jnp.full((R_BLK, LANES), jnp.inf, jnp.float32)
    cacc_ref[...] = jnp.zeros((R_BLK, LANES), jnp.float32)
    for c in range(K_CODES // C_CH):
        mm = lax.dot_general(
            flat, cb_vmem[pl.ds(c * C_CH, C_CH), :],
            (((1,), (1,)), ((), ())),
            preferred_element_type=jnp.float32,
        )
        dist = (rn - 2.0 * mm) + cn_ref[:, pl.ds(c * C_CH, C_CH)]
        d0 = dist[:, 0 * LANES:1 * LANES]
        d1 = dist[:, 1 * LANES:2 * LANES]
        d2 = dist[:, 2 * LANES:3 * LANES]
        d3 = dist[:, 3 * LANES:4 * LANES]
        c01 = d1 < d0
        m01 = jnp.minimum(d1, d0)
        g01 = jnp.where(c01, jnp.float32(4 * c + 1), jnp.float32(4 * c))
        c23 = d3 < d2
        m23 = jnp.minimum(d3, d2)
        g23 = jnp.where(c23, jnp.float32(4 * c + 3), jnp.float32(4 * c + 2))
        cc = m23 < m01
        m = jnp.minimum(m23, m01)
        g = jnp.where(cc, g23, g01)
        va = vacc_ref[...]
        u = m < va
        vacc_ref[...] = jnp.minimum(m, va)
        cacc_ref[...] = jnp.where(u, g, cacc_ref[...])
    vacc = vacc_ref[...]
    minval = jnp.min(vacc, axis=1, keepdims=True)
    lane = lax.broadcasted_iota(jnp.int32, (R_BLK, LANES), 1).astype(jnp.float32)
    fidx = cacc_ref[...] * jnp.float32(LANES) + lane
    ids = jnp.min(jnp.where(vacc == minval, fidx, jnp.float32(K_CODES)),
                  axis=1, keepdims=True)
    ids_ref[...] = ids.astype(jnp.int32)
    acc_ref[0] += jnp.sum(minval)

    @pl.when(i == pl.num_programs(0) - 1)
    def _():
        loss_ref[0] = acc_ref[0]


def _ids_and_loss(flat, codebook, cn):
    nb = flat.shape[0] // R_BLK
    ids2d, loss = pl.pallas_call(
        _dist_argmin_body,
        grid=(nb,),
        in_specs=[
            pl.BlockSpec((R_BLK, DIM), lambda i: (i, 0)),
            pl.BlockSpec(memory_space=pl.ANY),
            pl.BlockSpec((1, K_CODES), lambda i: (0, 0)),
        ],
        scratch_shapes=[
            pltpu.VMEM((K_CODES, DIM), jnp.float32),
            pltpu.SemaphoreType.DMA,
            pltpu.SMEM((1,), jnp.float32),
            pltpu.VMEM((R_BLK, LANES), jnp.float32),
            pltpu.VMEM((R_BLK, LANES), jnp.float32),
        ],
        out_specs=[
            pl.BlockSpec((R_BLK, 1), lambda i: (i, 0)),
            pl.BlockSpec(memory_space=pltpu.SMEM),
        ],
        out_shape=[
            jax.ShapeDtypeStruct((flat.shape[0], 1), jnp.int32),
            jax.ShapeDtypeStruct((1,), jnp.float32),
        ],
        compiler_params=pltpu.CompilerParams(
            dimension_semantics=("arbitrary",),
        ),
    )(flat, codebook, cn)
    return ids2d, loss


def _make_sc_gather(rows):
    info = plsc.get_sparse_core_info()
    nc, ns = info.num_cores, info.num_subcores
    nw = nc * ns
    bpw = rows // nw
    n_ch = (bpw + 127) // 128
    ch = bpw // n_ch
    mesh = plsc.VectorSubcoreMesh(core_axis_name="c", subcore_axis_name="s")

    @functools.partial(
        pl.kernel,
        mesh=mesh,
        out_type=jax.ShapeDtypeStruct((rows, DIM), jnp.float32),
        scratch_types=[
            pltpu.VMEM((n_ch, ch), jnp.int32),
            pltpu.VMEM((bpw, DIM), jnp.float32),
            pltpu.SemaphoreType.DMA,
        ],
        compiler_params=pltpu.CompilerParams(use_tc_tiling_on_sc=False),
    )
    def gather_k(table_hbm, idx_hbm, out_hbm, idx_v, rows_v, sem):
        wid = lax.axis_index("s") * nc + lax.axis_index("c")
        pltpu.sync_copy(idx_hbm.at[wid], idx_v)
        copies = [
            pltpu.async_copy(
                table_hbm.at[idx_v.at[j]],
                rows_v.at[pl.ds(j * ch, ch)],
                sem,
            )
            for j in range(n_ch)
        ]
        for c in copies:
            c.wait()
        pltpu.sync_copy(rows_v, out_hbm.at[pl.ds(wid * bpw, bpw)])

    return gather_k, nw, n_ch, ch


def kernel(z_e, codebook):
    B, T, D = z_e.shape
    flat = z_e.reshape(B * T, D)
    cn = jnp.sum(codebook ** 2, axis=1)[None, :]
    half = (B * T) // 2
    gather_k, nw, n_ch, ch = _make_sc_gather(half)
    ids_halves, zq_halves, losses = [], [], []
    for h in range(2):
        ids2d, loss = _ids_and_loss(flat[h * half:(h + 1) * half], codebook, cn)
        ids_h = ids2d.reshape(half)
        zq_halves.append(gather_k(codebook, ids_h.reshape(nw, n_ch, ch)))
        ids_halves.append(ids_h)
        losses.append(loss[0])

    ids = jnp.concatenate(ids_halves)
    z_q = jnp.concatenate(zq_halves)
    vq_loss = (1.25 / (B * T * D)) * (losses[0] + losses[1])
    return (z_q.reshape(B, T, D), ids.reshape(B, T), vq_loss)


# speedup vs baseline: 1.2572x; 1.2572x over previous
import functools

import jax
import jax.numpy as jnp
from jax import lax
from jax.experimental import pallas as pl
from jax.experimental.pallas import tpu as pltpu
from jax.experimental.pallas import tpu_sc as plsc

K_CODES = 8192
DIM = 64
ROWS = 9216
R_BLK = 1152
C_CH = 512
LANES = 128


def _dist_argmin_body(flat_ref, cb_hbm, cn_ref, ids_ref, loss_ref,
                      cb_vmem, cb_sem, acc_ref, vacc_ref, cacc_ref):
    i = pl.program_id(0)

    @pl.when(i == 0)
    def _():
        pltpu.make_async_copy(cb_hbm, cb_vmem, cb_sem).start()
        pltpu.make_async_copy(cb_hbm, cb_vmem, cb_sem).wait()
        acc_ref[0] = 0.0

    flat = flat_ref[...]
    rn = jnp.sum(flat ** 2, axis=1, keepdims=True)
    flat2 = flat + flat
    vacc_ref[...] = jnp.full((R_BLK, LANES), jnp.inf, jnp.float32)
    cacc_ref[...] = jnp.zeros((R_BLK, LANES), jnp.float32)
    for c in range(K_CODES // C_CH):
        mm2 = lax.dot_general(
            flat2, cb_vmem[pl.ds(c * C_CH, C_CH), :],
            (((1,), (1,)), ((), ())),
            preferred_element_type=jnp.float32,
        )
        dist = (rn - mm2) + cn_ref[:, pl.ds(c * C_CH, C_CH)]
        d0 = dist[:, 0 * LANES:1 * LANES]
        d1 = dist[:, 1 * LANES:2 * LANES]
        d2 = dist[:, 2 * LANES:3 * LANES]
        d3 = dist[:, 3 * LANES:4 * LANES]
        c01 = d1 < d0
        m01 = jnp.minimum(d1, d0)
        g01 = jnp.where(c01, jnp.float32(4 * c + 1), jnp.float32(4 * c))
        c23 = d3 < d2
        m23 = jnp.minimum(d3, d2)
        g23 = jnp.where(c23, jnp.float32(4 * c + 3), jnp.float32(4 * c + 2))
        cc = m23 < m01
        m = jnp.minimum(m23, m01)
        g = jnp.where(cc, g23, g01)
        va = vacc_ref[...]
        u = m < va
        vacc_ref[...] = jnp.minimum(m, va)
        cacc_ref[...] = jnp.where(u, g, cacc_ref[...])
    vacc = vacc_ref[...]
    minval = jnp.min(vacc, axis=1, keepdims=True)
    lane = lax.broadcasted_iota(jnp.int32, (R_BLK, LANES), 1).astype(jnp.float32)
    fidx = cacc_ref[...] * jnp.float32(LANES) + lane
    ids = jnp.min(jnp.where(vacc == minval, fidx, jnp.float32(K_CODES)),
                  axis=1, keepdims=True)
    ids_ref[...] = ids.astype(jnp.int32)
    acc_ref[0] += jnp.sum(minval)

    @pl.when(i == pl.num_programs(0) - 1)
    def _():
        loss_ref[0] = acc_ref[0]


def _ids_and_loss(flat, codebook, cn):
    nb = flat.shape[0] // R_BLK
    ids2d, loss = pl.pallas_call(
        _dist_argmin_body,
        grid=(nb,),
        in_specs=[
            pl.BlockSpec((R_BLK, DIM), lambda i: (i, 0)),
            pl.BlockSpec(memory_space=pl.ANY),
            pl.BlockSpec((1, K_CODES), lambda i: (0, 0)),
        ],
        scratch_shapes=[
            pltpu.VMEM((K_CODES, DIM), jnp.float32),
            pltpu.SemaphoreType.DMA,
            pltpu.SMEM((1,), jnp.float32),
            pltpu.VMEM((R_BLK, LANES), jnp.float32),
            pltpu.VMEM((R_BLK, LANES), jnp.float32),
        ],
        out_specs=[
            pl.BlockSpec((R_BLK, 1), lambda i: (i, 0)),
            pl.BlockSpec(memory_space=pltpu.SMEM),
        ],
        out_shape=[
            jax.ShapeDtypeStruct((flat.shape[0], 1), jnp.int32),
            jax.ShapeDtypeStruct((1,), jnp.float32),
        ],
        compiler_params=pltpu.CompilerParams(
            dimension_semantics=("arbitrary",),
        ),
    )(flat, codebook, cn)
    return ids2d, loss


def _make_sc_gather(rows):
    info = plsc.get_sparse_core_info()
    nc, ns = info.num_cores, info.num_subcores
    nw = nc * ns
    bpw = rows // nw
    n_ch = (bpw + 127) // 128
    ch = bpw // n_ch
    mesh = plsc.VectorSubcoreMesh(core_axis_name="c", subcore_axis_name="s")

    @functools.partial(
        pl.kernel,
        mesh=mesh,
        out_type=jax.ShapeDtypeStruct((rows, DIM), jnp.float32),
        scratch_types=[
            pltpu.VMEM((n_ch, ch), jnp.int32),
            pltpu.VMEM((bpw, DIM), jnp.float32),
            pltpu.SemaphoreType.DMA,
        ],
        compiler_params=pltpu.CompilerParams(use_tc_tiling_on_sc=False),
    )
    def gather_k(table_hbm, idx_hbm, out_hbm, idx_v, rows_v, sem):
        wid = lax.axis_index("s") * nc + lax.axis_index("c")
        pltpu.sync_copy(idx_hbm.at[wid], idx_v)
        copies = [
            pltpu.async_copy(
                table_hbm.at[idx_v.at[j]],
                rows_v.at[pl.ds(j * ch, ch)],
                sem,
            )
            for j in range(n_ch)
        ]
        for c in copies:
            c.wait()
        pltpu.sync_copy(rows_v, out_hbm.at[pl.ds(wid * bpw, bpw)])

    return gather_k, nw, n_ch, ch


def kernel(z_e, codebook):
    B, T, D = z_e.shape
    flat = z_e.reshape(B * T, D)
    cn = jnp.sum(codebook ** 2, axis=1)[None, :]
    ids2d, loss = _ids_and_loss(flat, codebook, cn)
    ids = ids2d.reshape(B * T)

    gather_k, nw, n_ch, ch = _make_sc_gather(B * T)
    z_q = gather_k(codebook, ids.reshape(nw, n_ch, ch))

    vq_loss = (1.25 / (B * T * D)) * loss[0]
    return (z_q.reshape(B, T, D), ids.reshape(B, T), vq_loss)
